# Initial kernel scaffold; baseline (speedup 1.0000x reference)
import jax
import jax.numpy as jnp
from jax.experimental import pallas as pl


def kernel(idx, table):
    raise NotImplementedError("write your pallas kernel here")


# speedup vs baseline: 1.0367x; 1.0367x over previous
import functools

import jax
import jax.numpy as jnp
from jax import lax
from jax.experimental import pallas as pl
from jax.experimental.pallas import tpu as pltpu
from jax.experimental.pallas import tpu_sc as plsc

NUM_WORKERS = 32
CHUNK = 40


@functools.lru_cache(maxsize=None)
def _make_sc_gather(B, D):
    b_per_w = B // NUM_WORKERS
    nchunk = b_per_w // CHUNK
    half = nchunk // 2
    mesh = plsc.VectorSubcoreMesh(core_axis_name="c", subcore_axis_name="s")

    @functools.partial(
        pl.kernel,
        mesh=mesh,
        compiler_params=pltpu.CompilerParams(use_tc_tiling_on_sc=False),
        out_type=jax.ShapeDtypeStruct((B, D), jnp.float32),
        scratch_types=[
            pltpu.VMEM((b_per_w,), jnp.int32),
            pltpu.VMEM((CHUNK, D), jnp.float32),
            pltpu.VMEM((CHUNK, D), jnp.float32),
            pltpu.SemaphoreType.DMA,
            pltpu.SemaphoreType.DMA,
        ],
    )
    def gather_kernel(idx_hbm, table_hbm, out_hbm, idx_v, buf0, buf1, gs0, gs1):
        wid = lax.axis_index("s") * 2 + lax.axis_index("c")
        base = wid * b_per_w
        pltpu.sync_copy(idx_hbm.at[pl.ds(base, b_per_w)], idx_v)
        pltpu.async_copy(table_hbm.at[idx_v.at[pl.ds(0, CHUNK)]], buf0, gs0)

        def step(j, carry):
            c0 = 2 * j
            c1 = c0 + 1
            pltpu.async_copy(
                table_hbm.at[idx_v.at[pl.ds(c1 * CHUNK, CHUNK)]], buf1, gs1)
            pltpu.make_async_copy(
                table_hbm.at[idx_v.at[pl.ds(c0 * CHUNK, CHUNK)]], buf0, gs0).wait()
            pltpu.sync_copy(buf0, out_hbm.at[pl.ds(base + c0 * CHUNK, CHUNK)])

            @pl.when(j < half - 1)
            def _():
                pltpu.async_copy(
                    table_hbm.at[idx_v.at[pl.ds((c0 + 2) * CHUNK, CHUNK)]], buf0, gs0)

            pltpu.make_async_copy(
                table_hbm.at[idx_v.at[pl.ds(c1 * CHUNK, CHUNK)]], buf1, gs1).wait()
            pltpu.sync_copy(buf1, out_hbm.at[pl.ds(base + c1 * CHUNK, CHUNK)])
            return carry

        lax.fori_loop(0, half, step, 0)

    return gather_kernel


def kernel(idx, table):
    batch, seq = idx.shape
    vocab, d = table.shape
    b_flat = batch * seq
    idx_flat = idx.reshape(b_flat).astype(jnp.int32)
    out = _make_sc_gather(b_flat, d)(idx_flat, table)
    return out.reshape(batch, seq, d)
